# baseline (speedup 1.0000x reference)
import functools

import jax
import jax.numpy as jnp
from jax import lax
from jax.experimental import pallas as pl
from jax.experimental.pallas import tpu as pltpu
from jax.experimental.pallas import tpu_sc as plsc

DIM = 32
CHB = 512
GD = 4
QUARTER = 4096

_info = plsc.get_sparse_core_info()
_NW = _info.num_cores * _info.num_subcores
_mesh = plsc.VectorSubcoreMesh(core_axis_name="c", subcore_axis_name="s")
_params = pltpu.CompilerParams(
    use_tc_tiling_on_sc=False, needs_layout_passes=False
)


def _wid():
    return lax.axis_index("s") * _info.num_cores + lax.axis_index("c")


def _iota16():
    return lax.broadcasted_iota(jnp.int32, (16,), 0)


@functools.cache
def _gather(n_b, n_l, dim):
    n_units = n_l * (n_b // QUARTER)
    per_w = n_units // _NW
    assert per_w * _NW == n_units and n_b % QUARTER == 0
    ch_per_u = QUARTER // CHB
    n_ch = per_w * ch_per_u
    g_per_ch = CHB // 128
    q_per_l = n_b // QUARTER
    d8 = dim // 8
    cbn = d8 * g_per_ch * 1024

    @functools.partial(
        pl.kernel,
        mesh=_mesh,
        out_type=jax.ShapeDtypeStruct((n_l, d8, (n_b // 128) * 1024), jnp.float32),
        scratch_types=[
            pltpu.VMEM((2, QUARTER), jnp.int32),
            pltpu.VMEM((GD, CHB, dim), jnp.float32),
            pltpu.VMEM((2, cbn), jnp.float32),
            pltpu.SemaphoreType.DMA,
            pltpu.SemaphoreType.DMA((GD,)),
            pltpu.SemaphoreType.DMA((2,)),
        ],
        compiler_params=_params,
    )
    def k(xt_hbm, tl_hbm, out_hbm, idx_v, rows_v, cb_v, isem, gsem, osem):
        w = _wid()
        u0 = w * per_w
        iota = _iota16()
        pre_lo = ((iota >> 3) << 12) + ((iota & 7) << 7)
        pre_hi = pre_lo + 2 * 4096

        def unit_coords(u):
            uid = u0 + u
            return uid // q_per_l, uid % q_per_l

        def stage_idx(u, sync):
            l, q = unit_coords(u)
            src = xt_hbm.at[l, pl.ds(q * QUARTER, QUARTER)]
            if sync:
                pltpu.sync_copy(src, idx_v.at[u % 2])
            else:
                pltpu.async_copy(src, idx_v.at[u % 2], isem)

        def start_gather(t):
            u, c = t // ch_per_u, t % ch_per_u
            idx = idx_v.at[u % 2, pl.ds(c * CHB, CHB)]
            pltpu.async_copy(tl_hbm.at[idx], rows_v.at[t % GD], gsem.at[t % GD])

        def out_dst(t, j):
            u, c = t // ch_per_u, t % ch_per_u
            l, q = unit_coords(u)
            off = (q * (QUARTER // 128) + c * g_per_ch) * 1024
            return out_hbm.at[l, j, pl.ds(off, g_per_ch * 1024)]

        stage_idx(0, True)
        stage_idx(1, False)
        for t0 in range(GD - 1):
            start_gather(t0)

        def body(t, _):
            p = t % 2

            tg = t + GD - 1
            boundary_g = tg % ch_per_u == 0
            nug = tg // ch_per_u

            @pl.when((tg < n_ch) & boundary_g)
            def _wait_idx():
                pltpu.make_async_copy(
                    xt_hbm.at[0, pl.ds(0, QUARTER)], idx_v.at[nug % 2], isem
                ).wait()

            @pl.when(tg < n_ch)
            def _next_gather():
                start_gather(tg)

            pltpu.make_async_copy(
                tl_hbm.at[idx_v.at[0, pl.ds(0, CHB)]],
                rows_v.at[t % GD],
                gsem.at[t % GD],
            ).wait()

            boundary = (t + 1) % ch_per_u == 0
            nu = (t + 1) // ch_per_u

            @pl.when(boundary & (nu + 1 < per_w))
            def _stage_next():
                stage_idx(nu + 1, False)

            @pl.when(t >= 2)
            def _drain_out():
                for j in range(d8):
                    pltpu.make_async_copy(
                        cb_v.at[p, pl.ds(j * g_per_ch * 1024, g_per_ch * 1024)],
                        out_dst(t, j),
                        osem.at[p],
                    ).wait()

            g = t % GD

            @plsc.parallel_loop(0, CHB, unroll=8)
            def _row(r):
                off_r = ((r >> 7) << 10) + (r & 127)
                vlo = rows_v[g, r, pl.ds(0, 16)]
                vhi = rows_v[g, r, pl.ds(16, 16)]
                plsc.store_scatter(cb_v.at[p], [pre_lo + off_r], vlo)
                plsc.store_scatter(cb_v.at[p], [pre_hi + off_r], vhi)

            for j in range(d8):
                pltpu.async_copy(
                    cb_v.at[p, pl.ds(j * g_per_ch * 1024, g_per_ch * 1024)],
                    out_dst(t, j),
                    osem.at[p],
                )

            return 0

        lax.fori_loop(0, n_ch, body, 0)

        for t in (n_ch - 2, n_ch - 1):
            for j in range(d8):
                pltpu.make_async_copy(
                    cb_v.at[t % 2, pl.ds(j * g_per_ch * 1024, g_per_ch * 1024)],
                    out_dst(t, j),
                    osem.at[t % 2],
                ).wait()

    return k


def kernel(x, table):
    b, l = x.shape
    vocab, dim = table.shape
    out3 = _gather(b, l, dim)(x.T, table)
    out5 = out3.reshape(l, dim // 8, b // 128, 8, 128)
    return jnp.transpose(out5, (2, 4, 0, 1, 3)).reshape(b, l, dim)


# speedup vs baseline: 1.0022x; 1.0022x over previous
import functools

import jax
import jax.numpy as jnp
from jax import lax
from jax.experimental import pallas as pl
from jax.experimental.pallas import tpu as pltpu
from jax.experimental.pallas import tpu_sc as plsc

DIM = 32
CHB = 512
GD = 4
QUARTER = 4096

_info = plsc.get_sparse_core_info()
_NW = _info.num_cores * _info.num_subcores
_mesh = plsc.VectorSubcoreMesh(core_axis_name="c", subcore_axis_name="s")
_params = pltpu.CompilerParams(
    use_tc_tiling_on_sc=False, needs_layout_passes=False
)


def _wid():
    return lax.axis_index("s") * _info.num_cores + lax.axis_index("c")


def _iota16():
    return lax.broadcasted_iota(jnp.int32, (16,), 0)


@functools.cache
def _gather(n_b, n_l, dim):
    n_units = n_l * (n_b // QUARTER)
    per_w = n_units // _NW
    assert per_w * _NW == n_units and n_b % QUARTER == 0
    ch_per_u = QUARTER // CHB
    n_ch = per_w * ch_per_u
    g_per_ch = CHB // 128
    q_per_l = n_b // QUARTER
    d8 = dim // 8
    cbn = d8 * g_per_ch * 1024

    @functools.partial(
        pl.kernel,
        mesh=_mesh,
        out_type=jax.ShapeDtypeStruct((n_l, d8, (n_b // 128) * 1024), jnp.float32),
        scratch_types=[
            pltpu.VMEM((2, QUARTER), jnp.int32),
            pltpu.VMEM((GD, CHB, dim), jnp.float32),
            pltpu.VMEM((2, d8, g_per_ch * 1024), jnp.float32),
            pltpu.SemaphoreType.DMA,
            pltpu.SemaphoreType.DMA((GD,)),
            pltpu.SemaphoreType.DMA((2,)),
        ],
        compiler_params=_params,
    )
    def k(xt_hbm, tl_hbm, out_hbm, idx_v, rows_v, cb_v, isem, gsem, osem):
        w = _wid()
        u0 = w * per_w
        iota = _iota16()
        d8_lo = iota >> 3
        d8_hi = d8_lo + 2
        pre_col = (iota & 7) << 7

        def unit_coords(u):
            uid = u0 + u
            return uid // q_per_l, uid % q_per_l

        def stage_idx(u, sync):
            l, q = unit_coords(u)
            src = xt_hbm.at[l, pl.ds(q * QUARTER, QUARTER)]
            if sync:
                pltpu.sync_copy(src, idx_v.at[u % 2])
            else:
                pltpu.async_copy(src, idx_v.at[u % 2], isem)

        def start_gather(t):
            u, c = t // ch_per_u, t % ch_per_u
            idx = idx_v.at[u % 2, pl.ds(c * CHB, CHB)]
            pltpu.async_copy(tl_hbm.at[idx], rows_v.at[t % GD], gsem.at[t % GD])

        def out_dst(t):
            u, c = t // ch_per_u, t % ch_per_u
            l, q = unit_coords(u)
            off = (q * (QUARTER // 128) + c * g_per_ch) * 1024
            return out_hbm.at[l, :, pl.ds(off, g_per_ch * 1024)]

        stage_idx(0, True)
        stage_idx(1, False)
        for t0 in range(GD - 1):
            start_gather(t0)

        def body(t, _):
            p = t % 2

            tg = t + GD - 1
            boundary_g = tg % ch_per_u == 0
            nug = tg // ch_per_u

            @pl.when((tg < n_ch) & boundary_g)
            def _wait_idx():
                pltpu.make_async_copy(
                    xt_hbm.at[0, pl.ds(0, QUARTER)], idx_v.at[nug % 2], isem
                ).wait()

            @pl.when(tg < n_ch)
            def _next_gather():
                start_gather(tg)

            pltpu.make_async_copy(
                tl_hbm.at[idx_v.at[0, pl.ds(0, CHB)]],
                rows_v.at[t % GD],
                gsem.at[t % GD],
            ).wait()

            boundary = (t + 1) % ch_per_u == 0
            nu = (t + 1) // ch_per_u

            @pl.when(boundary & (nu + 1 < per_w))
            def _stage_next():
                stage_idx(nu + 1, False)

            @pl.when(t >= 2)
            def _drain_out():
                pltpu.make_async_copy(
                    cb_v.at[p], out_dst(t), osem.at[p]
                ).wait()

            g = t % GD

            @plsc.parallel_loop(0, CHB, unroll=8)
            def _row(r):
                off_r = ((r >> 7) << 10) + (r & 127)
                vlo = rows_v[g, r, pl.ds(0, 16)]
                vhi = rows_v[g, r, pl.ds(16, 16)]
                col = pre_col + off_r
                plsc.store_scatter(cb_v.at[p], [d8_lo, col], vlo)
                plsc.store_scatter(cb_v.at[p], [d8_hi, col], vhi)

            pltpu.async_copy(cb_v.at[p], out_dst(t), osem.at[p])

            return 0

        lax.fori_loop(0, n_ch, body, 0)

        for t in (n_ch - 2, n_ch - 1):
            pltpu.make_async_copy(
                cb_v.at[t % 2], out_dst(t), osem.at[t % 2]
            ).wait()

    return k


def kernel(x, table):
    b, l = x.shape
    vocab, dim = table.shape
    out3 = _gather(b, l, dim)(x.T, table)
    out5 = out3.reshape(l, dim // 8, b // 128, 8, 128)
    return jnp.transpose(out5, (2, 4, 0, 1, 3)).reshape(b, l, dim)


# speedup vs baseline: 2.6969x; 2.6910x over previous
import functools

import jax
import jax.numpy as jnp
from jax import lax
from jax.experimental import pallas as pl
from jax.experimental.pallas import tpu as pltpu
from jax.experimental.pallas import tpu_sc as plsc

DIM = 32
CHB = 512
GD = 4
QUARTER = 4096

_info = plsc.get_sparse_core_info()
_NW = _info.num_cores * _info.num_subcores
_mesh = plsc.VectorSubcoreMesh(core_axis_name="c", subcore_axis_name="s")
_params = pltpu.CompilerParams(
    use_tc_tiling_on_sc=False, needs_layout_passes=False
)


def _wid():
    return lax.axis_index("s") * _info.num_cores + lax.axis_index("c")


def _iota16():
    return lax.broadcasted_iota(jnp.int32, (16,), 0)


@functools.cache
def _gather(n_b, n_l, dim):
    n_units = n_l * (n_b // QUARTER)
    per_w = n_units // _NW
    assert per_w * _NW == n_units and n_b % QUARTER == 0
    ch_per_u = QUARTER // CHB
    n_ch = per_w * ch_per_u
    g_per_ch = CHB // 128
    q_per_l = n_b // QUARTER
    d8 = dim // 8
    cbn = d8 * g_per_ch * 1024

    @functools.partial(
        pl.kernel,
        mesh=_mesh,
        out_type=jax.ShapeDtypeStruct((n_l, d8, n_b // 128, 8, 128), jnp.float32),
        scratch_types=[
            pltpu.VMEM((2, QUARTER), jnp.int32),
            pltpu.VMEM((GD, CHB, dim), jnp.float32),
            pltpu.VMEM((2, d8, g_per_ch, 8, 129), jnp.float32),
            pltpu.SemaphoreType.DMA,
            pltpu.SemaphoreType.DMA((GD,)),
            pltpu.SemaphoreType.DMA((2,)),
        ],
        compiler_params=_params,
    )
    def k(xt_hbm, tl_hbm, out_hbm, idx_v, rows_v, cb_v, isem, gsem, osem):
        w = _wid()
        u0 = w * per_w
        iota = _iota16()
        d8_lo = iota >> 3
        d8_hi = d8_lo + 2
        s8_v = iota & 7

        def unit_coords(u):
            uid = u0 + u
            return uid // q_per_l, uid % q_per_l

        def stage_idx(u, sync):
            l, q = unit_coords(u)
            src = xt_hbm.at[l, pl.ds(q * QUARTER, QUARTER)]
            if sync:
                pltpu.sync_copy(src, idx_v.at[u % 2])
            else:
                pltpu.async_copy(src, idx_v.at[u % 2], isem)

        def start_gather(t):
            u, c = t // ch_per_u, t % ch_per_u
            idx = idx_v.at[u % 2, pl.ds(c * CHB, CHB)]
            pltpu.async_copy(tl_hbm.at[idx], rows_v.at[t % GD], gsem.at[t % GD])

        def out_dst(t, j, gg):
            u, c = t // ch_per_u, t % ch_per_u
            l, q = unit_coords(u)
            bt = q * (QUARTER // 128) + c * g_per_ch + gg
            return out_hbm.at[l, j, bt]

        stage_idx(0, True)
        stage_idx(1, False)
        for t0 in range(GD - 1):
            start_gather(t0)

        def body(t, _):
            p = t % 2

            tg = t + GD - 1
            boundary_g = tg % ch_per_u == 0
            nug = tg // ch_per_u

            @pl.when((tg < n_ch) & boundary_g)
            def _wait_idx():
                pltpu.make_async_copy(
                    xt_hbm.at[0, pl.ds(0, QUARTER)], idx_v.at[nug % 2], isem
                ).wait()

            @pl.when(tg < n_ch)
            def _next_gather():
                start_gather(tg)

            pltpu.make_async_copy(
                tl_hbm.at[idx_v.at[0, pl.ds(0, CHB)]],
                rows_v.at[t % GD],
                gsem.at[t % GD],
            ).wait()

            boundary = (t + 1) % ch_per_u == 0
            nu = (t + 1) // ch_per_u

            @pl.when(boundary & (nu + 1 < per_w))
            def _stage_next():
                stage_idx(nu + 1, False)

            @pl.when(t >= 2)
            def _drain_out():
                for j in range(d8):
                    for gg in range(g_per_ch):
                        pltpu.make_async_copy(
                            cb_v.at[p, j, gg, :, pl.ds(0, 128)],
                            out_dst(t, j, gg),
                            osem.at[p],
                        ).wait()

            g = t % GD

            @plsc.parallel_loop(0, CHB, unroll=8)
            def _row(r):
                gv = jnp.full((16,), r >> 7, dtype=jnp.int32)
                lv = jnp.full((16,), r & 127, dtype=jnp.int32)
                vlo = rows_v[g, r, pl.ds(0, 16)]
                vhi = rows_v[g, r, pl.ds(16, 16)]
                plsc.store_scatter(cb_v.at[p], [d8_lo, gv, s8_v, lv], vlo)
                plsc.store_scatter(cb_v.at[p], [d8_hi, gv, s8_v, lv], vhi)

            for j in range(d8):
                for gg in range(g_per_ch):
                    pltpu.async_copy(
                        cb_v.at[p, j, gg, :, pl.ds(0, 128)],
                        out_dst(t, j, gg),
                        osem.at[p],
                    )

            return 0

        lax.fori_loop(0, n_ch, body, 0)

        for t in (n_ch - 2, n_ch - 1):
            for j in range(d8):
                for gg in range(g_per_ch):
                    pltpu.make_async_copy(
                        cb_v.at[t % 2, j, gg, :, pl.ds(0, 128)],
                        out_dst(t, j, gg),
                        osem.at[t % 2],
                    ).wait()

    return k


def kernel(x, table):
    b, l = x.shape
    vocab, dim = table.shape
    out5 = _gather(b, l, dim)(x.T, table)
    return jnp.transpose(out5, (2, 4, 0, 1, 3)).reshape(b, l, dim)
